# Initial kernel scaffold; baseline (speedup 1.0000x reference)
import jax
import jax.numpy as jnp
from jax.experimental import pallas as pl


def kernel(x, edge_index, W, b, gamma, beta):
    raise NotImplementedError("write your pallas kernel here")


# speedup vs baseline: 55.0506x; 55.0506x over previous
import functools

import jax
import jax.numpy as jnp
from jax import lax
from jax.experimental import pallas as pl
from jax.experimental.pallas import tpu as pltpu
from jax.experimental.pallas import tpu_sc as plsc

B, N, E, D = 2, 10000, 160000, 128
NC, NS = 2, 16
K = 128
ER = E // K
RA = 624
RLAST = N - (NS - 1) * RA


def _lin_body(x_ref, wt_ref, b_ref, o_ref):
    acc = jax.lax.dot_general(
        x_ref[...], wt_ref[...], (((1,), (0,)), ((), ())),
        precision=jax.lax.Precision.HIGHEST,
        preferred_element_type=jnp.float32)
    o_ref[...] = acc + b_ref[...]


def _linear(x2d, wt, b2d):
    BLK = 2000
    return pl.pallas_call(
        _lin_body,
        grid=(x2d.shape[0] // BLK,),
        in_specs=[
            pl.BlockSpec((BLK, D), lambda i: (i, 0)),
            pl.BlockSpec((D, D), lambda i: (0, 0)),
            pl.BlockSpec((1, D), lambda i: (0, 0)),
        ],
        out_specs=pl.BlockSpec((BLK, D), lambda i: (i, 0)),
        out_shape=jax.ShapeDtypeStruct((x2d.shape[0], D), jnp.float32),
    )(x2d, wt, b2d)


def _sc_body(xt_hbm, srcb_hbm, dst_hbm, z128_hbm, z1_hbm, ones_hbm,
             agg_hbm, deg_hbm,
             acc_sh, deg_sh, src_v, dst_v, rows_v, ones_v, sem):
    c = lax.axis_index("c")
    s = lax.axis_index("s")

    @pl.when(s < NS - 1)
    def _():
        pltpu.sync_copy(z128_hbm.at[pl.ds(0, RA)], acc_sh.at[pl.ds(s * RA, RA)])

        @pl.when(c == 0)
        def _():
            pltpu.sync_copy(z1_hbm.at[pl.ds(0, RA)], deg_sh.at[pl.ds(s * RA, RA)])

    @pl.when(s == NS - 1)
    def _():
        pltpu.sync_copy(z128_hbm, acc_sh.at[pl.ds((NS - 1) * RA, RLAST)])

        @pl.when(c == 0)
        def _():
            pltpu.sync_copy(z1_hbm, deg_sh.at[pl.ds((NS - 1) * RA, RLAST)])

    pltpu.sync_copy(ones_hbm, ones_v)
    plsc.subcore_barrier()

    nrows = jnp.where(s < ER - (ER // NS) * NS, ER // NS + 1, ER // NS)

    def chunk(i, carry):
        row = s + i * NS
        pltpu.sync_copy(srcb_hbm.at[pl.ds(c * ER + row, 1)], src_v)
        pltpu.sync_copy(dst_hbm.at[pl.ds(row, 1)], dst_v)
        pltpu.async_copy(xt_hbm.at[src_v.at[0]], rows_v, sem).wait()
        pltpu.sync_copy(rows_v, acc_sh.at[dst_v.at[0]], add=True)

        @pl.when(c == 0)
        def _():
            pltpu.sync_copy(ones_v, deg_sh.at[src_v.at[0]], add=True)

        return carry

    lax.fori_loop(0, nrows, chunk, 0)
    plsc.subcore_barrier()

    @pl.when(s < NS - 1)
    def _():
        rb = s * RA
        pltpu.sync_copy(acc_sh.at[pl.ds(rb, RA)],
                        agg_hbm.at[pl.ds(c * N + rb, RA)])

        @pl.when(c == 0)
        def _():
            pltpu.sync_copy(deg_sh.at[pl.ds(rb, RA)], deg_hbm.at[pl.ds(rb, RA)])

    @pl.when(s == NS - 1)
    def _():
        rb = (NS - 1) * RA
        pltpu.sync_copy(acc_sh.at[pl.ds(rb, RLAST)],
                        agg_hbm.at[pl.ds(c * N + rb, RLAST)])

        @pl.when(c == 0)
        def _():
            pltpu.sync_copy(deg_sh.at[pl.ds(rb, RLAST)],
                            deg_hbm.at[pl.ds(rb, RLAST)])


def _scatter(xt_flat, srcb, dst, z128, z1, ones):
    mesh = plsc.VectorSubcoreMesh(core_axis_name="c", subcore_axis_name="s")
    f = pl.kernel(
        _sc_body,
        out_type=(
            jax.ShapeDtypeStruct((B * N, D), jnp.float32),
            jax.ShapeDtypeStruct((N, 8), jnp.float32),
        ),
        mesh=mesh,
        scratch_types=[
            pltpu.VMEM_SHARED((N, D), jnp.float32),
            pltpu.VMEM_SHARED((N, 8), jnp.float32),
            pltpu.VMEM((1, K), jnp.int32),
            pltpu.VMEM((1, K), jnp.int32),
            pltpu.VMEM((K, D), jnp.float32),
            pltpu.VMEM((K, 8), jnp.float32),
            pltpu.SemaphoreType.DMA,
        ],
        compiler_params=pltpu.CompilerParams(use_tc_tiling_on_sc=False),
    )
    return f(xt_flat, srcb, dst, z128, z1, ones)


def _epi_body(xt_ref, agg_ref, deg_ref, gs_ref, bt_ref, o_ref):
    d = jnp.maximum(deg_ref[..., 0:1], 1.0)
    v = agg_ref[0] / d + xt_ref[0]
    o_ref[0] = jnp.maximum(v * gs_ref[...] + bt_ref[...], 0.0)


def _epilogue(xt3, agg3, deg, gs2, bt2):
    BLK = 2000
    return pl.pallas_call(
        _epi_body,
        grid=(B, N // BLK),
        in_specs=[
            pl.BlockSpec((1, BLK, D), lambda b, i: (b, i, 0)),
            pl.BlockSpec((1, BLK, D), lambda b, i: (b, i, 0)),
            pl.BlockSpec((BLK, 8), lambda b, i: (i, 0)),
            pl.BlockSpec((1, D), lambda b, i: (0, 0)),
            pl.BlockSpec((1, D), lambda b, i: (0, 0)),
        ],
        out_specs=pl.BlockSpec((1, BLK, D), lambda b, i: (b, i, 0)),
        out_shape=jax.ShapeDtypeStruct((B, N, D), jnp.float32),
    )(xt3, agg3, deg, gs2, bt2)


@jax.jit
def kernel(x, edge_index, W, b, gamma, beta):
    x2d = x.reshape(B * N, D)
    xt_flat = _linear(x2d, W.T, b[None, :])

    src = edge_index[0]
    dst = edge_index[1].reshape(ER, K)
    srcb = jnp.concatenate([src, src + N]).reshape(2 * ER, K)
    z128 = jnp.zeros((RLAST, D), jnp.float32)
    z1 = jnp.zeros((RLAST, 8), jnp.float32)
    ones = jnp.ones((K, 8), jnp.float32)

    agg_flat, deg = _scatter(xt_flat, srcb, dst, z128, z1, ones)

    eps = 1e-5
    gs = gamma / jnp.sqrt(1.0 + eps)
    out = _epilogue(xt_flat.reshape(B, N, D), agg_flat.reshape(B, N, D),
                    deg, gs[None, :], beta[None, :])
    return out


# speedup vs baseline: 86.1285x; 1.5645x over previous
import functools

import jax
import jax.numpy as jnp
from jax import lax
from jax.experimental import pallas as pl
from jax.experimental.pallas import tpu as pltpu
from jax.experimental.pallas import tpu_sc as plsc

B, N, E, D = 2, 10000, 160000, 128
NC, NS = 2, 16
K = 128
ER = E // K
J = ER // NS
LEFT = ER - J * NS
RA = 624
RLAST = N - (NS - 1) * RA


def _lin_body(x_ref, wt_ref, b_ref, o_ref):
    acc = jax.lax.dot_general(
        x_ref[...], wt_ref[...], (((1,), (0,)), ((), ())),
        precision=jax.lax.Precision.HIGHEST,
        preferred_element_type=jnp.float32)
    o_ref[...] = acc + b_ref[...]


def _linear(x2d, wt, b2d):
    BLK = 2000
    return pl.pallas_call(
        _lin_body,
        grid=(x2d.shape[0] // BLK,),
        in_specs=[
            pl.BlockSpec((BLK, D), lambda i: (i, 0)),
            pl.BlockSpec((D, D), lambda i: (0, 0)),
            pl.BlockSpec((1, D), lambda i: (0, 0)),
        ],
        out_specs=pl.BlockSpec((BLK, D), lambda i: (i, 0)),
        out_shape=jax.ShapeDtypeStruct((x2d.shape[0], D), jnp.float32),
    )(x2d, wt, b2d)


def _sc_body(xt_hbm, comb_hbm, z128_hbm, z1_hbm, ones_hbm,
             agg_hbm, deg_hbm,
             acc_sh, deg_sh, idx_v, rows_v, ones_v, sem0, sem1):
    c = lax.axis_index("c")
    s = lax.axis_index("s")
    sems = (sem0, sem1)

    @pl.when(s < NS - 1)
    def _():
        pltpu.sync_copy(z128_hbm.at[pl.ds(0, RA)], acc_sh.at[pl.ds(s * RA, RA)])

        @pl.when(c == 0)
        def _():
            pltpu.sync_copy(z1_hbm.at[pl.ds(0, RA)], deg_sh.at[pl.ds(s * RA, RA)])

    @pl.when(s == NS - 1)
    def _():
        pltpu.sync_copy(z128_hbm, acc_sh.at[pl.ds((NS - 1) * RA, RLAST)])

        @pl.when(c == 0)
        def _():
            pltpu.sync_copy(z1_hbm, deg_sh.at[pl.ds((NS - 1) * RA, RLAST)])

    pltpu.sync_copy(ones_hbm, ones_v)
    plsc.subcore_barrier()


    def load_idx(row, t):
        pltpu.sync_copy(comb_hbm.at[pl.ds((c * ER + row) * 2, 2)],
                        idx_v.at[pl.ds(2 * t, 2)])

    def fire_gather(t):
        return pltpu.async_copy(xt_hbm.at[idx_v.at[2 * t]],
                                rows_v.at[pl.ds(t * K, K)], sems[t])

    def drain_gather(t):
        pltpu.make_async_copy(xt_hbm.at[pl.ds(0, K)],
                              rows_v.at[pl.ds(t * K, K)], sems[t]).wait()

    def scatter(t):
        pltpu.sync_copy(rows_v.at[pl.ds(t * K, K)],
                        acc_sh.at[idx_v.at[2 * t + 1]], add=True)

        @pl.when(c == 0)
        def _():
            pltpu.sync_copy(ones_v, deg_sh.at[idx_v.at[2 * t]], add=True)

    load_idx(s, 0)
    fire_gather(0)

    def pair(i, carry):
        for t in (0, 1):
            j = 2 * i + t
            nxt = 1 - t

            @pl.when(j + 1 < J)
            def _():
                load_idx(s + (j + 1) * NS, nxt)
                fire_gather(nxt)

            drain_gather(t)
            scatter(t)
        return carry

    lax.fori_loop(0, (J + 1) // 2, pair, 0)

    @pl.when(s < LEFT)
    def _():
        load_idx(J * NS + s, 0)
        fire_gather(0)
        drain_gather(0)
        scatter(0)

    plsc.subcore_barrier()

    @pl.when(s < NS - 1)
    def _():
        rb = s * RA
        pltpu.sync_copy(acc_sh.at[pl.ds(rb, RA)],
                        agg_hbm.at[pl.ds(c * N + rb, RA)])

        @pl.when(c == 0)
        def _():
            pltpu.sync_copy(deg_sh.at[pl.ds(rb, RA)], deg_hbm.at[pl.ds(rb, RA)])

    @pl.when(s == NS - 1)
    def _():
        rb = (NS - 1) * RA
        pltpu.sync_copy(acc_sh.at[pl.ds(rb, RLAST)],
                        agg_hbm.at[pl.ds(c * N + rb, RLAST)])

        @pl.when(c == 0)
        def _():
            pltpu.sync_copy(deg_sh.at[pl.ds(rb, RLAST)],
                            deg_hbm.at[pl.ds(rb, RLAST)])


def _scatter(xt_flat, comb, z128, z1, ones):
    mesh = plsc.VectorSubcoreMesh(core_axis_name="c", subcore_axis_name="s")
    f = pl.kernel(
        _sc_body,
        out_type=(
            jax.ShapeDtypeStruct((B * N, D), jnp.float32),
            jax.ShapeDtypeStruct((N, 8), jnp.float32),
        ),
        mesh=mesh,
        scratch_types=[
            pltpu.VMEM_SHARED((N, D), jnp.float32),
            pltpu.VMEM_SHARED((N, 8), jnp.float32),
            pltpu.VMEM((4, K), jnp.int32),
            pltpu.VMEM((2 * K, D), jnp.float32),
            pltpu.VMEM((K, 8), jnp.float32),
            pltpu.SemaphoreType.DMA,
            pltpu.SemaphoreType.DMA,
        ],
        compiler_params=pltpu.CompilerParams(use_tc_tiling_on_sc=False),
    )
    return f(xt_flat, comb, z128, z1, ones)


def _epi_body(xt_ref, agg_ref, deg_ref, gs_ref, bt_ref, o_ref):
    d = jnp.maximum(deg_ref[..., 0:1], 1.0)
    v = agg_ref[0] / d + xt_ref[0]
    o_ref[0] = jnp.maximum(v * gs_ref[...] + bt_ref[...], 0.0)


def _epilogue(xt3, agg3, deg, gs2, bt2):
    BLK = 2000
    return pl.pallas_call(
        _epi_body,
        grid=(B, N // BLK),
        in_specs=[
            pl.BlockSpec((1, BLK, D), lambda b, i: (b, i, 0)),
            pl.BlockSpec((1, BLK, D), lambda b, i: (b, i, 0)),
            pl.BlockSpec((BLK, 8), lambda b, i: (i, 0)),
            pl.BlockSpec((1, D), lambda b, i: (0, 0)),
            pl.BlockSpec((1, D), lambda b, i: (0, 0)),
        ],
        out_specs=pl.BlockSpec((1, BLK, D), lambda b, i: (b, i, 0)),
        out_shape=jax.ShapeDtypeStruct((B, N, D), jnp.float32),
    )(xt3, agg3, deg, gs2, bt2)


@jax.jit
def kernel(x, edge_index, W, b, gamma, beta):
    x2d = x.reshape(B * N, D)
    xt_flat = _linear(x2d, W.T, b[None, :])

    src2d = edge_index[0].reshape(ER, K)
    dst2d = edge_index[1].reshape(ER, K)
    comb = jnp.concatenate(
        [jnp.stack([src2d + c * N, dst2d], axis=1).reshape(2 * ER, K)
         for c in range(B)])
    z128 = jnp.zeros((RLAST, D), jnp.float32)
    z1 = jnp.zeros((RLAST, 8), jnp.float32)
    ones = jnp.ones((K, 8), jnp.float32)

    agg_flat, deg = _scatter(xt_flat, comb, z128, z1, ones)

    eps = 1e-5
    gs = gamma / jnp.sqrt(1.0 + eps)
    out = _epilogue(xt_flat.reshape(B, N, D), agg_flat.reshape(B, N, D),
                    deg, gs[None, :], beta[None, :])
    return out


# speedup vs baseline: 86.3194x; 1.0022x over previous
import functools

import jax
import jax.numpy as jnp
from jax import lax
from jax.experimental import pallas as pl
from jax.experimental.pallas import tpu as pltpu
from jax.experimental.pallas import tpu_sc as plsc

B, N, E, D = 2, 10000, 160000, 128
NC, NS = 2, 16
K = 128
ER = E // K
J = ER // NS
LEFT = ER - J * NS
RA = 624
RLAST = N - (NS - 1) * RA


def _lin_body(x_ref, wt_ref, b_ref, o_ref):
    acc = jax.lax.dot_general(
        x_ref[...], wt_ref[...], (((1,), (0,)), ((), ())),
        precision=jax.lax.Precision.HIGHEST,
        preferred_element_type=jnp.float32)
    o_ref[...] = acc + b_ref[...]


def _linear(x2d, wt, b2d):
    BLK = 2000
    return pl.pallas_call(
        _lin_body,
        grid=(x2d.shape[0] // BLK,),
        in_specs=[
            pl.BlockSpec((BLK, D), lambda i: (i, 0)),
            pl.BlockSpec((D, D), lambda i: (0, 0)),
            pl.BlockSpec((1, D), lambda i: (0, 0)),
        ],
        out_specs=pl.BlockSpec((BLK, D), lambda i: (i, 0)),
        out_shape=jax.ShapeDtypeStruct((x2d.shape[0], D), jnp.float32),
    )(x2d, wt, b2d)


def _sc_body(xt_hbm, comb_hbm, z128_hbm, z1_hbm, ones_hbm,
             agg_hbm, deg_hbm,
             acc_sh, deg_sh, idx_v, rows_v, ones_v, sem0, sem1):
    c = lax.axis_index("c")
    s = lax.axis_index("s")
    sems = (sem0, sem1)

    @pl.when(s < NS - 1)
    def _():
        pltpu.sync_copy(z128_hbm.at[pl.ds(0, RA)], acc_sh.at[pl.ds(s * RA, RA)])

        @pl.when(c == 0)
        def _():
            pltpu.sync_copy(z1_hbm.at[pl.ds(0, RA)], deg_sh.at[pl.ds(s * RA, RA)])

    @pl.when(s == NS - 1)
    def _():
        pltpu.sync_copy(z128_hbm, acc_sh.at[pl.ds((NS - 1) * RA, RLAST)])

        @pl.when(c == 0)
        def _():
            pltpu.sync_copy(z1_hbm, deg_sh.at[pl.ds((NS - 1) * RA, RLAST)])

    pltpu.sync_copy(ones_hbm, ones_v)
    plsc.subcore_barrier()


    def load_idx(row, t):
        pltpu.sync_copy(comb_hbm.at[pl.ds((c * ER + row) * 2, 2)],
                        idx_v.at[pl.ds(2 * t, 2)])

    def fire_gather(t):
        return pltpu.async_copy(xt_hbm.at[idx_v.at[2 * t]],
                                rows_v.at[pl.ds(t * K, K)], sems[t])

    def drain_gather(t):
        pltpu.make_async_copy(xt_hbm.at[pl.ds(0, K)],
                              rows_v.at[pl.ds(t * K, K)], sems[t]).wait()

    def scatter(t):
        pltpu.sync_copy(rows_v.at[pl.ds(t * K, K)],
                        acc_sh.at[idx_v.at[2 * t + 1]], add=True)

        @pl.when(c == 0)
        def _():
            pltpu.sync_copy(ones_v, deg_sh.at[idx_v.at[2 * t]], add=True)

    load_idx(s, 0)
    fire_gather(0)

    def pair(i, carry):
        for t in (0, 1):
            j = 2 * i + t
            nxt = 1 - t

            @pl.when(j + 1 < J)
            def _():
                load_idx(s + (j + 1) * NS, nxt)
                fire_gather(nxt)

            drain_gather(t)
            scatter(t)
        return carry

    lax.fori_loop(0, (J + 1) // 2, pair, 0)

    @pl.when(s < LEFT)
    def _():
        load_idx(J * NS + s, 0)
        fire_gather(0)
        drain_gather(0)
        scatter(0)

    plsc.subcore_barrier()

    @pl.when(s < NS - 1)
    def _():
        rb = s * RA
        pltpu.sync_copy(acc_sh.at[pl.ds(rb, RA)],
                        agg_hbm.at[pl.ds(c * N + rb, RA)])

        @pl.when(c == 0)
        def _():
            pltpu.sync_copy(deg_sh.at[pl.ds(rb, RA)], deg_hbm.at[pl.ds(rb, RA)])

    @pl.when(s == NS - 1)
    def _():
        rb = (NS - 1) * RA
        pltpu.sync_copy(acc_sh.at[pl.ds(rb, RLAST)],
                        agg_hbm.at[pl.ds(c * N + rb, RLAST)])

        @pl.when(c == 0)
        def _():
            pltpu.sync_copy(deg_sh.at[pl.ds(rb, RLAST)],
                            deg_hbm.at[pl.ds(rb, RLAST)])


def _scatter(xt_flat, comb, z128, z1, ones):
    mesh = plsc.VectorSubcoreMesh(core_axis_name="c", subcore_axis_name="s")
    f = pl.kernel(
        _sc_body,
        out_type=(
            jax.ShapeDtypeStruct((B * N, D), jnp.float32),
            jax.ShapeDtypeStruct((N, 8), jnp.float32),
        ),
        mesh=mesh,
        scratch_types=[
            pltpu.VMEM_SHARED((N, D), jnp.float32),
            pltpu.VMEM_SHARED((N, 8), jnp.float32),
            pltpu.VMEM((4, K), jnp.int32),
            pltpu.VMEM((2 * K, D), jnp.float32),
            pltpu.VMEM((K, 8), jnp.float32),
            pltpu.SemaphoreType.DMA,
            pltpu.SemaphoreType.DMA,
        ],
        compiler_params=pltpu.CompilerParams(use_tc_tiling_on_sc=False),
    )
    return f(xt_flat, comb, z128, z1, ones)


def _epi_body(xt_ref, agg_ref, deg_ref, gs_ref, bt_ref, o_ref):
    d = jnp.maximum(deg_ref[..., 0:1], 1.0)
    v = agg_ref[0] / d + xt_ref[0]
    o_ref[0] = jnp.maximum(v * gs_ref[...] + bt_ref[...], 0.0)


def _epilogue(xt3, agg3, deg2, gs2, bt2):
    BLK = 2000
    return pl.pallas_call(
        _epi_body,
        grid=(B, N // BLK),
        in_specs=[
            pl.BlockSpec((1, BLK, D), lambda b, i: (b, i, 0)),
            pl.BlockSpec((1, BLK, D), lambda b, i: (b, i, 0)),
            pl.BlockSpec((BLK, 8), lambda b, i: (i, 0)),
            pl.BlockSpec((1, D), lambda b, i: (0, 0)),
            pl.BlockSpec((1, D), lambda b, i: (0, 0)),
        ],
        out_specs=pl.BlockSpec((1, BLK, D), lambda b, i: (b, i, 0)),
        out_shape=jax.ShapeDtypeStruct((B, N, D), jnp.float32),
    )(xt3, agg3, deg2, gs2, bt2)


@jax.jit
def kernel(x, edge_index, W, b, gamma, beta):
    x2d = x.reshape(B * N, D)
    xt_flat = _linear(x2d, W.T, b[None, :])

    src2d = edge_index[0].reshape(ER, K)
    dst2d = edge_index[1].reshape(ER, K)
    comb = jnp.concatenate(
        [jnp.stack([src2d + c * N, dst2d], axis=1).reshape(2 * ER, K)
         for c in range(B)])
    z128 = jnp.zeros((RLAST, D), jnp.float32)
    z1 = jnp.zeros((RLAST, 8), jnp.float32)
    ones = jnp.ones((K, 8), jnp.float32)

    agg_flat, deg = _scatter(xt_flat, comb, z128, z1, ones)

    eps = 1e-5
    gs = gamma / jnp.sqrt(1.0 + eps)
    out = _epilogue(xt_flat.reshape(B, N, D), agg_flat.reshape(B, N, D),
                    deg, gs[None, :], beta[None, :])
    return out
